# baseline (speedup 1.0000x reference)
import functools

import jax
import jax.numpy as jnp
from jax.experimental import pallas as pl
from jax.experimental.pallas import tpu as pltpu


def _fused_kernel(emb_ref, wg_ref, bg_ref, w1_ref, b1_ref, w2_ref, b2_ref,
                  projs_ref, gate_ref, *, n_experts: int, proj_dim: int):
    x = emb_ref[...]

    gt = jax.lax.dot_general(
        wg_ref[...], x, (((0,), (1,)), ((), ())),
        preferred_element_type=jnp.float32) + bg_ref[...]
    rows = [gt[i:i + 1, :] for i in range(n_experts)]
    adj = []
    for i in range(n_experts):
        rank = None
        for j in range(n_experts):
            if j == i:
                continue
            beats = rows[j] > rows[i]
            if j < i:
                beats = beats | (rows[j] == rows[i])
            b = beats.astype(jnp.float32)
            rank = b if rank is None else rank + b
        mask = (rank < 2.0).astype(jnp.float32)
        adj.append(rows[i] + jnp.log(mask + 1e-9))
    m = adj[0]
    for i in range(1, n_experts):
        m = jnp.maximum(m, adj[i])
    ex = [jnp.exp(a - m) for a in adj]
    s = ex[0]
    for i in range(1, n_experts):
        s = s + ex[i]
    gate_ref[...] = jnp.concatenate([e / s for e in ex], axis=0)

    h = jax.lax.dot_general(
        x, w1_ref[...], (((1,), (0,)), ((), ())),
        preferred_element_type=jnp.float32) + b1_ref[...]
    h = jnp.maximum(h, 0.0).astype(jnp.bfloat16)
    for e in range(n_experts):
        he = h[:, e * proj_dim:(e + 1) * proj_dim]
        out = jax.lax.dot_general(
            he, w2_ref[e], (((1,), (0,)), ((), ())),
            preferred_element_type=jnp.float32)
        projs_ref[:, e * proj_dim:(e + 1) * proj_dim] = (
            out + b2_ref[:, e * proj_dim:(e + 1) * proj_dim])


def kernel(emb, Wg, bg, W1, b1, W2, b2, top_k):
    del top_k
    B, in_dim = emb.shape
    E = Wg.shape[1]
    P = W2.shape[-1]

    BT = min(1024, B)
    assert B % BT == 0

    W1f = W1.transpose(1, 0, 2).reshape(in_dim, E * P).astype(jnp.bfloat16)
    b1f = b1.reshape(1, E * P)
    b2f = b2.reshape(1, E * P)
    bg2 = bg.reshape(E, 1)
    emb_bf = emb.astype(jnp.bfloat16)
    Wg_bf = Wg.astype(jnp.bfloat16)
    W2_bf = W2.astype(jnp.bfloat16)

    grid = (B // BT,)
    projs2d, gate_w = pl.pallas_call(
        functools.partial(_fused_kernel, n_experts=E, proj_dim=P),
        grid=grid,
        in_specs=[
            pl.BlockSpec((BT, in_dim), lambda i: (i, 0)),
            pl.BlockSpec((in_dim, E), lambda i: (0, 0)),
            pl.BlockSpec((E, 1), lambda i: (0, 0)),
            pl.BlockSpec((in_dim, E * P), lambda i: (0, 0)),
            pl.BlockSpec((1, E * P), lambda i: (0, 0)),
            pl.BlockSpec((E, P, P), lambda i: (0, 0, 0)),
            pl.BlockSpec((1, E * P), lambda i: (0, 0)),
        ],
        out_specs=[
            pl.BlockSpec((BT, E * P), lambda i: (i, 0)),
            pl.BlockSpec((E, BT), lambda i: (0, i)),
        ],
        out_shape=[
            jax.ShapeDtypeStruct((B, E * P), jnp.float32),
            jax.ShapeDtypeStruct((E, B), jnp.float32),
        ],
        compiler_params=pltpu.CompilerParams(
            dimension_semantics=("arbitrary",)),
    )(emb_bf, Wg_bf, bg2, W1f, b1f, W2_bf, b2f)

    return projs2d.reshape(B, E, P), gate_w.T


# speedup vs baseline: 1.2030x; 1.2030x over previous
import functools

import jax
import jax.numpy as jnp
from jax.experimental import pallas as pl
from jax.experimental.pallas import tpu as pltpu


def _fused_kernel(emb_ref, wg_ref, bg_ref, w1_ref, b1_ref, w2_ref, b2_ref,
                  projs_ref, gate_ref, *, n_experts: int, proj_dim: int):
    x = emb_ref[...]

    gt = jax.lax.dot_general(
        wg_ref[...], x, (((0,), (1,)), ((), ())),
        preferred_element_type=jnp.float32) + bg_ref[...]
    rows = [gt[i:i + 1, :] for i in range(n_experts)]
    adj = []
    for i in range(n_experts):
        rank = None
        for j in range(n_experts):
            if j == i:
                continue
            beats = rows[j] > rows[i]
            if j < i:
                beats = beats | (rows[j] == rows[i])
            b = beats.astype(jnp.float32)
            rank = b if rank is None else rank + b
        mask = (rank < 2.0).astype(jnp.float32)
        adj.append(rows[i] + jnp.log(mask + 1e-9))
    m = adj[0]
    for i in range(1, n_experts):
        m = jnp.maximum(m, adj[i])
    ex = [jnp.exp(a - m) for a in adj]
    s = ex[0]
    for i in range(1, n_experts):
        s = s + ex[i]
    gate_ref[...] = jnp.concatenate([e / s for e in ex], axis=0)

    h = jax.lax.dot_general(
        x, w1_ref[...], (((1,), (0,)), ((), ())),
        preferred_element_type=jnp.float32) + b1_ref[...]
    h = jnp.maximum(h, 0.0)
    for e in range(n_experts):
        he = h[:, e * proj_dim:(e + 1) * proj_dim]
        out = jax.lax.dot_general(
            he, w2_ref[e], (((1,), (0,)), ((), ())),
            preferred_element_type=jnp.float32)
        projs_ref[:, e * proj_dim:(e + 1) * proj_dim] = (
            out + b2_ref[:, e * proj_dim:(e + 1) * proj_dim])


def kernel(emb, Wg, bg, W1, b1, W2, b2, top_k):
    del top_k
    B, in_dim = emb.shape
    E = Wg.shape[1]
    P = W2.shape[-1]

    BT = min(1024, B)
    assert B % BT == 0

    W1f = W1.transpose(1, 0, 2).reshape(in_dim, E * P)
    b1f = b1.reshape(1, E * P)
    b2f = b2.reshape(1, E * P)
    bg2 = bg.reshape(E, 1)

    grid = (B // BT,)
    projs2d, gate_w = pl.pallas_call(
        functools.partial(_fused_kernel, n_experts=E, proj_dim=P),
        grid=grid,
        in_specs=[
            pl.BlockSpec((BT, in_dim), lambda i: (i, 0)),
            pl.BlockSpec((in_dim, E), lambda i: (0, 0)),
            pl.BlockSpec((E, 1), lambda i: (0, 0)),
            pl.BlockSpec((in_dim, E * P), lambda i: (0, 0)),
            pl.BlockSpec((1, E * P), lambda i: (0, 0)),
            pl.BlockSpec((E, P, P), lambda i: (0, 0, 0)),
            pl.BlockSpec((1, E * P), lambda i: (0, 0)),
        ],
        out_specs=[
            pl.BlockSpec((BT, E * P), lambda i: (i, 0)),
            pl.BlockSpec((E, BT), lambda i: (0, i)),
        ],
        out_shape=[
            jax.ShapeDtypeStruct((B, E * P), jnp.float32),
            jax.ShapeDtypeStruct((E, B), jnp.float32),
        ],
        compiler_params=pltpu.CompilerParams(
            dimension_semantics=("parallel",)),
    )(emb, Wg, bg2, W1f, b1f, W2, b2f)

    return projs2d.reshape(B, E, P), gate_w.T
